# Initial kernel scaffold; baseline (speedup 1.0000x reference)
import jax
import jax.numpy as jnp
from jax.experimental import pallas as pl


def kernel(x, edge_index, W1, b1, core_w, core_b, W2, b2):
    raise NotImplementedError("write your pallas kernel here")


# speedup vs baseline: 4.2363x; 4.2363x over previous
import functools

import jax
import jax.numpy as jnp
from jax import lax
from jax.experimental import pallas as pl
from jax.experimental.pallas import tpu as pltpu
from jax.experimental.pallas import tpu_sc as plsc

ROW_W = 8
NUM_CORES = 2
NUM_SUBCORES = 16
NUM_TILES = NUM_CORES * NUM_SUBCORES
CHUNK = 128


def _signals_tc(x, cw8, cb8, blk):
    n, f = x.shape

    def body(x_ref, w_ref, b_ref, o_ref):
        z = jnp.dot(x_ref[...], w_ref[...],
                    preferred_element_type=jnp.float32) + b_ref[...]
        t = jnp.tanh(z)
        col = lax.broadcasted_iota(jnp.int32, (blk, ROW_W), 1)
        o_ref[...] = jnp.where(col == 4, 1.0, jnp.where(col > 4, 0.0, t))

    return pl.pallas_call(
        body,
        grid=(n // blk,),
        in_specs=[
            pl.BlockSpec((blk, f), lambda i: (i, 0)),
            pl.BlockSpec((f, ROW_W), lambda i: (0, 0)),
            pl.BlockSpec((1, ROW_W), lambda i: (0, 0)),
        ],
        out_specs=pl.BlockSpec((blk, ROW_W), lambda i: (i, 0)),
        out_shape=jax.ShapeDtypeStruct((n, ROW_W), jnp.float32),
    )(x, cw8, cb8)


def _segment_sums_sc(t_pad, src3, dst3, zeros):
    n_pad = t_pad.shape[0]
    nch = src3.shape[1]
    rows_per_sub = n_pad // NUM_SUBCORES
    mesh = plsc.VectorSubcoreMesh(core_axis_name="c", subcore_axis_name="s",
                                  num_cores=NUM_CORES,
                                  num_subcores=NUM_SUBCORES)

    @functools.partial(
        pl.kernel,
        out_type=jax.ShapeDtypeStruct((NUM_CORES, n_pad, ROW_W), jnp.float32),
        mesh=mesh,
        scratch_types=[
            pltpu.VMEM((nch, CHUNK), jnp.int32),
            pltpu.VMEM((nch, CHUNK), jnp.int32),
            pltpu.VMEM((CHUNK, ROW_W), jnp.float32),
            pltpu.VMEM_SHARED((n_pad, ROW_W), jnp.float32),
        ],
        compiler_params=pltpu.CompilerParams(use_tc_tiling_on_sc=False),
    )
    def k(t_hbm, src_hbm, dst_hbm, zero_hbm, out_hbm,
          src_v, dst_v, rows_v, accum_sh):
        c = lax.axis_index("c")
        s = lax.axis_index("s")
        wid = c * NUM_SUBCORES + s
        pltpu.sync_copy(src_hbm.at[wid], src_v)
        pltpu.sync_copy(dst_hbm.at[wid], dst_v)
        base = s * rows_per_sub
        pltpu.sync_copy(zero_hbm, accum_sh.at[pl.ds(base, rows_per_sub)])
        plsc.subcore_barrier()

        def chunk(j, carry):
            pltpu.sync_copy(t_hbm.at[src_v.at[j]], rows_v)
            pltpu.sync_copy(rows_v, accum_sh.at[dst_v.at[j]], add=True)
            return carry

        lax.fori_loop(0, nch, chunk, 0)
        plsc.subcore_barrier()
        pltpu.sync_copy(accum_sh.at[pl.ds(base, rows_per_sub)],
                        out_hbm.at[c, pl.ds(base, rows_per_sub)])

    return k(t_pad, src3, dst3, zeros)


def _output_tc(partial, xt, w2, b2r, blk):
    n, o = xt.shape
    n_pad = partial.shape[1]

    def body(p_ref, x_ref, w_ref, b_ref, o_ref):
        p = p_ref[0] + p_ref[1]
        cnt = p[:, 4:5]
        avg = jnp.where(cnt > 0.0, p[:, 0:4] / jnp.maximum(cnt, 1.0), 0.0)
        w = w_ref[...]
        u = (avg[:, 0:1] * w[0:1, :] + avg[:, 1:2] * w[1:2, :]
             + avg[:, 2:3] * w[2:3, :] + avg[:, 3:4] * w[3:4, :])
        out = x_ref[...][:, :, None] * u[:, None, :] + b_ref[...][None, :, :]
        o_ref[...] = jnp.maximum(out, 0.0)

    return pl.pallas_call(
        body,
        grid=(n // blk,),
        in_specs=[
            pl.BlockSpec((NUM_CORES, blk, ROW_W), lambda i: (0, i, 0)),
            pl.BlockSpec((blk, o), lambda i: (i, 0)),
            pl.BlockSpec((4, o), lambda i: (0, 0)),
            pl.BlockSpec((1, o), lambda i: (0, 0)),
        ],
        out_specs=pl.BlockSpec((blk, o, o), lambda i: (i, 0, 0)),
        out_shape=jax.ShapeDtypeStruct((n, o, o), jnp.float32),
    )(partial, xt, w2, b2r)


def kernel(x, edge_index, W1, b1, core_w, core_b, W2, b2):
    n, f = x.shape
    c = core_w.shape[0]
    o = W2.shape[1]
    e = edge_index.shape[1]

    cw8 = jnp.zeros((f, ROW_W), x.dtype).at[:, :c].set(core_w.T)
    cb8 = jnp.zeros((1, ROW_W), x.dtype).at[0, :c].set(core_b)
    t = _signals_tc(x, cw8, cb8, blk=400)

    n_pad = -(-n // (NUM_SUBCORES * 16)) * (NUM_SUBCORES * 16)
    t_pad = jnp.zeros((n_pad, ROW_W), x.dtype).at[:n].set(t)

    src = edge_index[0].astype(jnp.int32)
    dst = edge_index[1].astype(jnp.int32)
    nch = -(-e // (NUM_TILES * CHUNK))
    e_pad = NUM_TILES * nch * CHUNK
    src3 = jnp.concatenate(
        [src, jnp.full((e_pad - e,), n, jnp.int32)]).reshape(
            NUM_TILES, nch, CHUNK)
    dst3 = jnp.concatenate(
        [dst, jnp.zeros((e_pad - e,), jnp.int32)]).reshape(
            NUM_TILES, nch, CHUNK)
    zeros = jnp.zeros((n_pad // NUM_SUBCORES, ROW_W), jnp.float32)
    partial = _segment_sums_sc(t_pad, src3, dst3, zeros)

    xt = x[:, :o]
    return _output_tc(partial, xt, W2, b2.reshape(1, o), blk=400)


# speedup vs baseline: 4.9707x; 1.1734x over previous
import functools

import jax
import jax.numpy as jnp
from jax import lax
from jax.experimental import pallas as pl
from jax.experimental.pallas import tpu as pltpu
from jax.experimental.pallas import tpu_sc as plsc

ROW_W = 8
NUM_CORES = 2
NUM_SUBCORES = 16
NUM_TILES = NUM_CORES * NUM_SUBCORES
CHUNK = 128


def _signals_tc(x, cw8, cb8, blk):
    n, f = x.shape

    def body(x_ref, w_ref, b_ref, o_ref):
        z = jnp.dot(x_ref[...], w_ref[...],
                    preferred_element_type=jnp.float32) + b_ref[...]
        t = jnp.tanh(z)
        col = lax.broadcasted_iota(jnp.int32, (blk, ROW_W), 1)
        o_ref[...] = jnp.where(col == 4, 1.0, jnp.where(col > 4, 0.0, t))

    return pl.pallas_call(
        body,
        grid=(n // blk,),
        in_specs=[
            pl.BlockSpec((blk, f), lambda i: (i, 0)),
            pl.BlockSpec((f, ROW_W), lambda i: (0, 0)),
            pl.BlockSpec((1, ROW_W), lambda i: (0, 0)),
        ],
        out_specs=pl.BlockSpec((blk, ROW_W), lambda i: (i, 0)),
        out_shape=jax.ShapeDtypeStruct((n, ROW_W), jnp.float32),
    )(x, cw8, cb8)


def _segment_sums_sc(t_pad, src3, dst3, zeros):
    n_pad = t_pad.shape[0]
    nch = src3.shape[1]
    rows_per_sub = n_pad // NUM_SUBCORES
    mesh = plsc.VectorSubcoreMesh(core_axis_name="c", subcore_axis_name="s",
                                  num_cores=NUM_CORES,
                                  num_subcores=NUM_SUBCORES)

    @functools.partial(
        pl.kernel,
        out_type=jax.ShapeDtypeStruct((NUM_CORES, n_pad, ROW_W), jnp.float32),
        mesh=mesh,
        scratch_types=[
            pltpu.VMEM((nch, CHUNK), jnp.int32),
            pltpu.VMEM((nch, CHUNK), jnp.int32),
            pltpu.VMEM((CHUNK, ROW_W), jnp.float32),
            pltpu.VMEM_SHARED((n_pad, ROW_W), jnp.float32),
        ],
        compiler_params=pltpu.CompilerParams(use_tc_tiling_on_sc=False),
    )
    def k(t_hbm, src_hbm, dst_hbm, zero_hbm, out_hbm,
          src_v, dst_v, rows_v, accum_sh):
        c = lax.axis_index("c")
        s = lax.axis_index("s")
        wid = c * NUM_SUBCORES + s
        pltpu.sync_copy(src_hbm.at[wid], src_v)
        pltpu.sync_copy(dst_hbm.at[wid], dst_v)
        base = s * rows_per_sub
        pltpu.sync_copy(zero_hbm, accum_sh.at[pl.ds(base, rows_per_sub)])
        plsc.subcore_barrier()

        def chunk(j, carry):
            pltpu.sync_copy(t_hbm.at[src_v.at[j]], rows_v)
            pltpu.sync_copy(rows_v, accum_sh.at[dst_v.at[j]], add=True)
            return carry

        lax.fori_loop(0, nch, chunk, 0)
        plsc.subcore_barrier()
        pltpu.sync_copy(accum_sh.at[pl.ds(base, rows_per_sub)],
                        out_hbm.at[c, pl.ds(base, rows_per_sub)])

    return k(t_pad, src3, dst3, zeros)


def _output_tc(partial, xt, w2, b2r, blk):
    n, o = xt.shape
    n_pad = partial.shape[1]

    def body(p_ref, x_ref, w_ref, b_ref, o_ref):
        p = p_ref[0] + p_ref[1]
        cnt = p[:, 4:5]
        avg = jnp.where(cnt > 0.0, p[:, 0:4] / jnp.maximum(cnt, 1.0), 0.0)
        w = w_ref[...]
        u = (avg[:, 0:1] * w[0:1, :] + avg[:, 1:2] * w[1:2, :]
             + avg[:, 2:3] * w[2:3, :] + avg[:, 3:4] * w[3:4, :])
        out = x_ref[...][:, :, None] * u[:, None, :] + b_ref[...][None, :, :]
        o_ref[...] = jnp.maximum(out, 0.0).reshape(blk, o * o)

    out2d = pl.pallas_call(
        body,
        grid=(n // blk,),
        in_specs=[
            pl.BlockSpec((NUM_CORES, blk, ROW_W), lambda i: (0, i, 0)),
            pl.BlockSpec((blk, o), lambda i: (i, 0)),
            pl.BlockSpec((4, o), lambda i: (0, 0)),
            pl.BlockSpec((1, o), lambda i: (0, 0)),
        ],
        out_specs=pl.BlockSpec((blk, o * o), lambda i: (i, 0)),
        out_shape=jax.ShapeDtypeStruct((n, o * o), jnp.float32),
    )(partial, xt, w2, b2r)
    return out2d.reshape(n, o, o)


def kernel(x, edge_index, W1, b1, core_w, core_b, W2, b2):
    n, f = x.shape
    c = core_w.shape[0]
    o = W2.shape[1]
    e = edge_index.shape[1]

    cw8 = jnp.zeros((f, ROW_W), x.dtype).at[:, :c].set(core_w.T)
    cb8 = jnp.zeros((1, ROW_W), x.dtype).at[0, :c].set(core_b)
    t = _signals_tc(x, cw8, cb8, blk=400)

    n_pad = -(-n // (NUM_SUBCORES * 16)) * (NUM_SUBCORES * 16)
    t_pad = jnp.zeros((n_pad, ROW_W), x.dtype).at[:n].set(t)

    src = edge_index[0].astype(jnp.int32)
    dst = edge_index[1].astype(jnp.int32)
    nch = -(-e // (NUM_TILES * CHUNK))
    e_pad = NUM_TILES * nch * CHUNK
    src3 = jnp.concatenate(
        [src, jnp.full((e_pad - e,), n, jnp.int32)]).reshape(
            NUM_TILES, nch, CHUNK)
    dst3 = jnp.concatenate(
        [dst, jnp.zeros((e_pad - e,), jnp.int32)]).reshape(
            NUM_TILES, nch, CHUNK)
    zeros = jnp.zeros((n_pad // NUM_SUBCORES, ROW_W), jnp.float32)
    partial = _segment_sums_sc(t_pad, src3, dst3, zeros)

    xt = x[:, :o]
    return _output_tc(partial, xt, W2, b2.reshape(1, o), blk=400)


# speedup vs baseline: 9.9155x; 1.9948x over previous
import functools

import jax
import jax.numpy as jnp
from jax import lax
from jax.experimental import pallas as pl
from jax.experimental.pallas import tpu as pltpu
from jax.experimental.pallas import tpu_sc as plsc

ROW_W = 8
NUM_CORES = 2
NUM_SUBCORES = 16
NUM_TILES = NUM_CORES * NUM_SUBCORES
CHUNK = 128


def _signals_tc(x, cw8, cb8, blk):
    n, f = x.shape

    def body(x_ref, w_ref, b_ref, o_ref):
        z = jnp.dot(x_ref[...], w_ref[...],
                    preferred_element_type=jnp.float32) + b_ref[...]
        t = jnp.tanh(z)
        col = lax.broadcasted_iota(jnp.int32, (blk, ROW_W), 1)
        o_ref[...] = jnp.where(col == 4, 1.0, jnp.where(col > 4, 0.0, t))

    return pl.pallas_call(
        body,
        grid=(n // blk,),
        in_specs=[
            pl.BlockSpec((blk, f), lambda i: (i, 0)),
            pl.BlockSpec((f, ROW_W), lambda i: (0, 0)),
            pl.BlockSpec((1, ROW_W), lambda i: (0, 0)),
        ],
        out_specs=pl.BlockSpec((blk, ROW_W), lambda i: (i, 0)),
        out_shape=jax.ShapeDtypeStruct((n, ROW_W), jnp.float32),
    )(x, cw8, cb8)


def _segment_sums_sc(t_pad, src3, dst3, zeros):
    n_pad = t_pad.shape[0]
    nch = src3.shape[1]
    rows_per_sub = n_pad // NUM_SUBCORES
    mesh = plsc.VectorSubcoreMesh(core_axis_name="c", subcore_axis_name="s",
                                  num_cores=NUM_CORES,
                                  num_subcores=NUM_SUBCORES)

    @functools.partial(
        pl.kernel,
        out_type=jax.ShapeDtypeStruct((NUM_CORES, n_pad, ROW_W), jnp.float32),
        mesh=mesh,
        scratch_types=[
            pltpu.VMEM((nch, CHUNK), jnp.int32),
            pltpu.VMEM((nch, CHUNK), jnp.int32),
            pltpu.VMEM((CHUNK, ROW_W), jnp.float32),
            pltpu.VMEM_SHARED((n_pad, ROW_W), jnp.float32),
        ],
        compiler_params=pltpu.CompilerParams(use_tc_tiling_on_sc=False),
    )
    def k(t_hbm, src_hbm, dst_hbm, zero_hbm, out_hbm,
          src_v, dst_v, rows_v, accum_sh):
        c = lax.axis_index("c")
        s = lax.axis_index("s")
        wid = c * NUM_SUBCORES + s
        pltpu.sync_copy(src_hbm.at[wid], src_v)
        pltpu.sync_copy(dst_hbm.at[wid], dst_v)
        base = s * rows_per_sub
        pltpu.sync_copy(zero_hbm, accum_sh.at[pl.ds(base, rows_per_sub)])
        plsc.subcore_barrier()

        def chunk(j, carry):
            pltpu.sync_copy(t_hbm.at[src_v.at[j]], rows_v)
            pltpu.sync_copy(rows_v, accum_sh.at[dst_v.at[j]], add=True)
            return carry

        lax.fori_loop(0, nch, chunk, 0)
        plsc.subcore_barrier()
        pltpu.sync_copy(accum_sh.at[pl.ds(base, rows_per_sub)],
                        out_hbm.at[c, pl.ds(base, rows_per_sub)])

    return k(t_pad, src3, dst3, zeros)


def _output_tc(partial_t, xt_t, w2, b2c, blkf):
    o, n = xt_t.shape

    blko = 32

    def body(p_ref, x_ref, w_ref, b_ref, o_ref):
        p = p_ref[0] + p_ref[1]
        cnt = p[4:5, :]
        avg = jnp.where(cnt > 0.0, p[0:4, :] / jnp.maximum(cnt, 1.0), 0.0)
        w = w_ref[...]
        u = (w[:, 0:1] * avg[0:1, :] + w[:, 1:2] * avg[1:2, :]
             + w[:, 2:3] * avg[2:3, :] + w[:, 3:4] * avg[3:4, :])
        out = x_ref[...][:, None, :] * u[None, :, :] + b_ref[...][None, :, :]
        o_ref[...] = jnp.maximum(out, 0.0)

    out_t = pl.pallas_call(
        body,
        grid=(o // blkf, o // blko),
        in_specs=[
            pl.BlockSpec((NUM_CORES, ROW_W, n), lambda i, j: (0, 0, 0)),
            pl.BlockSpec((blkf, n), lambda i, j: (i, 0)),
            pl.BlockSpec((blko, 4), lambda i, j: (j, 0)),
            pl.BlockSpec((blko, 1), lambda i, j: (j, 0)),
        ],
        out_specs=pl.BlockSpec((blkf, blko, n), lambda i, j: (i, j, 0)),
        out_shape=jax.ShapeDtypeStruct((o, o, n), jnp.float32),
    )(partial_t, xt_t, w2, b2c)
    return out_t.transpose(2, 0, 1)


def kernel(x, edge_index, W1, b1, core_w, core_b, W2, b2):
    n, f = x.shape
    c = core_w.shape[0]
    o = W2.shape[1]
    e = edge_index.shape[1]

    cw8 = jnp.zeros((f, ROW_W), x.dtype).at[:, :c].set(core_w.T)
    cb8 = jnp.zeros((1, ROW_W), x.dtype).at[0, :c].set(core_b)
    t = _signals_tc(x, cw8, cb8, blk=400)

    n_pad = -(-n // (NUM_SUBCORES * 16)) * (NUM_SUBCORES * 16)
    t_pad = jnp.zeros((n_pad, ROW_W), x.dtype).at[:n].set(t)

    src = edge_index[0].astype(jnp.int32)
    dst = edge_index[1].astype(jnp.int32)
    nch = -(-e // (NUM_TILES * CHUNK))
    e_pad = NUM_TILES * nch * CHUNK
    src3 = jnp.concatenate(
        [src, jnp.full((e_pad - e,), n, jnp.int32)]).reshape(
            NUM_TILES, nch, CHUNK)
    dst3 = jnp.concatenate(
        [dst, jnp.zeros((e_pad - e,), jnp.int32)]).reshape(
            NUM_TILES, nch, CHUNK)
    zeros = jnp.zeros((n_pad // NUM_SUBCORES, ROW_W), jnp.float32)
    partial = _segment_sums_sc(t_pad, src3, dst3, zeros)

    xt_t = x[:, :o].T
    partial_t = partial[:, :n, :].transpose(0, 2, 1)
    return _output_tc(partial_t, xt_t, W2.T, b2.reshape(o, 1), blkf=8)
